# baseline (speedup 1.0000x reference)
import functools

import jax
import jax.numpy as jnp
from jax import lax
from jax.experimental import pallas as pl
from jax.experimental.pallas import tpu as pltpu
from jax.experimental.pallas import tpu_sc as plsc

_B, _N, _G, _K = 16, 8192, 256, 32
_BIG = 1 << 30


_CH = 2048
_NCH = _N // _CH


def _tc_dist_body(x_ref, c_ref, dist_ref):
    cmat = c_ref[0]
    xs = x_ref[0]
    d0 = cmat[:, 0:1] - xs[0:1, :]
    d1 = cmat[:, 1:2] - xs[1:2, :]
    d2 = cmat[:, 2:3] - xs[2:3, :]
    dist_ref[0] = (d0 * d0 + d1 * d1) + d2 * d2


_DIST_KW = dict(
    grid=(_B, _NCH),
    in_specs=[
        pl.BlockSpec((1, 8, _CH), lambda b, c: (b, 0, c)),
        pl.BlockSpec((1, _G, 8), lambda b, c: (b, 0, 0)),
    ],
    out_specs=pl.BlockSpec((1, _G, _CH), lambda b, c: (b, 0, c)),
    out_shape=jax.ShapeDtypeStruct((_B, _G, _N), jnp.float32),
)


_NW = 32
_RR = _B * _G
_RPW = _RR // _NW
_VPB = 2
_BLK = 16 * _VPB
_NBLK = _N // _BLK
_D = 16
_TW = 128


def _sc_topk_body(dist_hbm, tab_hbm, cen_hbm, out_hbm, idx_hbm,
                  dbuf, bmbuf, bval, bidx, cents, gbuf, grows, sem, gsem):
    w = lax.axis_index("s") * 2 + lax.axis_index("c")
    base_row = w * _RPW
    pltpu.sync_copy(cen_hbm.at[pl.ds(base_row, _RPW)], cents)
    lane = lax.iota(jnp.int32, 16)
    inf = jnp.float32(jnp.inf)

    def row_body(r, carry0):
        rr = base_row + r
        gbase = (rr // _G) * _N
        pltpu.sync_copy(dist_hbm.at[rr], dbuf)
        for t in range(3):
            bval[pl.ds(16 * t, 16)] = jnp.full((16,), inf, jnp.float32)
            bidx[pl.ds(16 * t, 16)] = jnp.full((16,), _BIG, jnp.int32)

        def seed_body(j, ms):
            me, mo = ms

            def bmin(blk):
                vs = [dbuf[pl.ds(blk * _BLK + 16 * i, 16)]
                      for i in range(_VPB)]
                bm = vs[0]
                for v in vs[1:]:
                    bm = jnp.minimum(bm, v)
                return bm

            bme = bmin(2 * j)
            bmo = bmin(2 * j + 1)
            bmbuf[pl.ds(2 * j * 16, 16)] = bme
            bmbuf[pl.ds((2 * j + 1) * 16, 16)] = bmo
            return (jnp.minimum(me, bme), jnp.minimum(mo, bmo))

        inf16 = jnp.full((16,), inf, jnp.float32)
        me0, mo0 = lax.fori_loop(0, _NBLK // 2, seed_body, (inf16, inf16))
        thr0 = jnp.max(jnp.maximum(me0, mo0))

        def blk_body(blk, thr_c):
            bms = jnp.min(bmbuf[pl.ds(blk * 16, 16)])

            def hit(tc):
                thr2, thri2 = tc
                for i in range(_VPB):
                    v = dbuf[pl.ds(blk * _BLK + 16 * i, 16)]
                    vbase = gbase + blk * _BLK + 16 * i
                    ivec = vbase + lane
                    cm = (v < thr2) | ((v == thr2) & (ivec < thri2))
                    cnt = jnp.sum(cm.astype(jnp.int32))

                    def wcond(st):
                        return st[1] > 0

                    def wbody(st):
                        cm2, c_, t_, ti_ = st
                        pos = jnp.min(jnp.where(cm2, lane, 16))
                        sv = jnp.max(jnp.where(lane == pos, v, -inf))
                        si = vbase + pos
                        b0 = bval[pl.ds(0, 16)]
                        b1 = bval[pl.ds(16, 16)]
                        i0 = bidx[pl.ds(0, 16)]
                        i1 = bidx[pl.ds(16, 16)]
                        lt0 = (b0 < sv) | ((b0 == sv) & (i0 < si))
                        lt1 = (b1 < sv) | ((b1 == sv) & (i1 < si))
                        p = (jnp.sum(lt0.astype(jnp.int32)) +
                             jnp.sum(lt1.astype(jnp.int32)))
                        sh0 = lane + (lane >= p).astype(jnp.int32)
                        sh1 = 16 + lane + ((16 + lane) >= p).astype(
                            jnp.int32)
                        plsc.store_scatter(bval, [sh0], b0)
                        plsc.store_scatter(bval, [sh1], b1)
                        plsc.store_scatter(bidx, [sh0], i0)
                        plsc.store_scatter(bidx, [sh1], i1)
                        m0 = lane == 0
                        pvec = jnp.broadcast_to(p, (16,))
                        plsc.store_scatter(
                            bval, [pvec], jnp.broadcast_to(sv, (16,)),
                            mask=m0)
                        plsc.store_scatter(
                            bidx, [pvec], jnp.broadcast_to(si, (16,)),
                            mask=m0)
                        tnew = jnp.where(p <= 30, b1[14], sv)
                        tinew = jnp.where(p <= 30, i1[14], si)
                        tv = jnp.minimum(tnew, thr0)
                        ti = jnp.where(tnew <= thr0, tinew, jnp.int32(_BIG))
                        return (cm2 & (lane != pos), c_ - 1, tv, ti)

                    thr2, thri2 = lax.while_loop(wcond, wbody,
                                                 (cm, cnt, thr2, thri2))[2:]
                return (thr2, thri2)

            return lax.cond(bms <= thr_c[0], hit, lambda tc: tc, thr_c)

        lax.fori_loop(0, _NBLK, blk_body, (thr0, jnp.int32(_BIG)))

        pltpu.sync_copy(bidx.at[pl.ds(0, _K)], idx_hbm.at[rr])
        pltpu.async_copy(tab_hbm.at[bidx.at[pl.ds(0, _K)]], gbuf, gsem).wait()
        cvec = cents[r]
        for i in range(_K):
            grows[i] = gbuf[i, pl.ds(0, _D)] - cvec
        pltpu.sync_copy(grows, out_hbm.at[pl.ds(rr * _K, _K)])
        return carry0

    lax.fori_loop(0, _RPW, row_body, 0)


_sc_topk = pl.kernel(
    _sc_topk_body,
    out_type=[
        jax.ShapeDtypeStruct((_RR * _K, _D), jnp.float32),
        jax.ShapeDtypeStruct((_RR, _K), jnp.int32),
    ],
    mesh=plsc.VectorSubcoreMesh(core_axis_name="c", subcore_axis_name="s"),
    compiler_params=pltpu.CompilerParams(use_tc_tiling_on_sc=False,
                                         needs_layout_passes=False),
    scratch_types=[
        pltpu.VMEM((_N,), jnp.float32),
        pltpu.VMEM((_NBLK * 16,), jnp.float32),
        pltpu.VMEM((48,), jnp.float32),
        pltpu.VMEM((48,), jnp.int32),
        pltpu.VMEM((_RPW, _D), jnp.float32),
        pltpu.VMEM((_K, _TW), jnp.float32),
        pltpu.VMEM((_K, _D), jnp.float32),
        pltpu.SemaphoreType.DMA,
        pltpu.SemaphoreType.DMA,
    ],
)


@jax.jit
def _run(xyz, center):
    xp = jnp.pad(xyz, ((0, 0), (0, 0), (0, 5))).transpose(0, 2, 1)
    cp = jnp.pad(center, ((0, 0), (0, 0), (0, 5)))
    dist = pl.pallas_call(_tc_dist_body, **_DIST_KW)(xp, cp)

    tab = jnp.pad(xyz.reshape(_B * _N, 3), ((0, 0), (0, _TW - 3)))
    cen = jnp.pad(center.reshape(_RR, 3), ((0, 0), (0, _D - 3)))
    out, idx = _sc_topk(dist.reshape(_RR, _N), tab, cen)
    neighborhood = out[:, :3].reshape(_B, _G, _K, 3)
    return neighborhood, idx.reshape(-1)


def kernel(xyz, center):
    neighborhood, idx_flat = _run(xyz, center)
    return (neighborhood, center, idx_flat)


# speedup vs baseline: 1.1470x; 1.1470x over previous
import functools

import jax
import jax.numpy as jnp
from jax import lax
from jax.experimental import pallas as pl
from jax.experimental.pallas import tpu as pltpu
from jax.experimental.pallas import tpu_sc as plsc

_B, _N, _G, _K = 16, 8192, 256, 32
_BIG = 1 << 30


_CH = 2048
_NCH = _N // _CH


def _tc_dist_body(x_ref, c_ref, dist_ref):
    cmat = c_ref[0]
    xs = x_ref[0]
    d0 = cmat[:, 0:1] - xs[0:1, :]
    d1 = cmat[:, 1:2] - xs[1:2, :]
    d2 = cmat[:, 2:3] - xs[2:3, :]
    dist_ref[0] = (d0 * d0 + d1 * d1) + d2 * d2


_DIST_KW = dict(
    grid=(_B, _NCH),
    in_specs=[
        pl.BlockSpec((1, 8, _CH), lambda b, c: (b, 0, c)),
        pl.BlockSpec((1, _G, 8), lambda b, c: (b, 0, 0)),
    ],
    out_specs=pl.BlockSpec((1, _G, _CH), lambda b, c: (b, 0, c)),
    out_shape=jax.ShapeDtypeStruct((_B, _G, _N), jnp.float32),
)


_NW = 32
_RR = _B * _G
_RPW = _RR // _NW
_VPB = 4
_BLK = 16 * _VPB
_NBLK = _N // _BLK
_D = 16
_TW = 128


def _sc_topk_body(dist_hbm, tab_hbm, cen_hbm, out_hbm, idx_hbm,
                  dbuf, dbuf2, bmbuf, bval, bidx, cents, gbuf, grows,
                  sem, sem2, gsem):
    w = lax.axis_index("s") * 2 + lax.axis_index("c")
    base_row = w * _RPW
    pltpu.sync_copy(cen_hbm.at[pl.ds(base_row, _RPW)], cents)
    lane = lax.iota(jnp.int32, 16)
    inf = jnp.float32(jnp.inf)

    def scan_row(rr, dv):
        gbase = (rr // _G) * _N
        for t in range(3):
            bval[pl.ds(16 * t, 16)] = jnp.full((16,), inf, jnp.float32)
            bidx[pl.ds(16 * t, 16)] = jnp.full((16,), _BIG, jnp.int32)

        def seed_body(j, ms):
            me, mo = ms

            def bmin(blk):
                vs = [dv[pl.ds(blk * _BLK + 16 * i, 16)]
                      for i in range(_VPB)]
                bm = vs[0]
                for v in vs[1:]:
                    bm = jnp.minimum(bm, v)
                return bm

            bme = bmin(2 * j)
            bmo = bmin(2 * j + 1)
            bmbuf[pl.ds(2 * j * 16, 16)] = bme
            bmbuf[pl.ds((2 * j + 1) * 16, 16)] = bmo
            return (jnp.minimum(me, bme), jnp.minimum(mo, bmo))

        inf16 = jnp.full((16,), inf, jnp.float32)
        me0, mo0 = lax.fori_loop(0, _NBLK // 2, seed_body, (inf16, inf16))
        thr0 = jnp.max(jnp.maximum(me0, mo0))

        def blk_body(blk, thr_c):
            bms = jnp.min(bmbuf[pl.ds(blk * 16, 16)])

            def hit(tc):
                thr2, thri2 = tc
                for i in range(_VPB):
                    v = dv[pl.ds(blk * _BLK + 16 * i, 16)]
                    vbase = gbase + blk * _BLK + 16 * i
                    ivec = vbase + lane
                    cm = (v < thr2) | ((v == thr2) & (ivec < thri2))
                    cnt = jnp.sum(cm.astype(jnp.int32))

                    def wcond(st):
                        return st[1] > 0

                    def wbody(st):
                        cm2, c_, t_, ti_ = st
                        pos = jnp.min(jnp.where(cm2, lane, 16))
                        sv = jnp.max(jnp.where(lane == pos, v, -inf))
                        si = vbase + pos
                        b0 = bval[pl.ds(0, 16)]
                        b1 = bval[pl.ds(16, 16)]
                        i0 = bidx[pl.ds(0, 16)]
                        i1 = bidx[pl.ds(16, 16)]
                        lt0 = (b0 < sv) | ((b0 == sv) & (i0 < si))
                        lt1 = (b1 < sv) | ((b1 == sv) & (i1 < si))
                        p = (jnp.sum(lt0.astype(jnp.int32)) +
                             jnp.sum(lt1.astype(jnp.int32)))
                        sh0 = lane + (lane >= p).astype(jnp.int32)
                        sh1 = 16 + lane + ((16 + lane) >= p).astype(
                            jnp.int32)
                        plsc.store_scatter(bval, [sh0], b0)
                        plsc.store_scatter(bval, [sh1], b1)
                        plsc.store_scatter(bidx, [sh0], i0)
                        plsc.store_scatter(bidx, [sh1], i1)
                        m0 = lane == 0
                        pvec = jnp.broadcast_to(p, (16,))
                        plsc.store_scatter(
                            bval, [pvec], jnp.broadcast_to(sv, (16,)),
                            mask=m0)
                        plsc.store_scatter(
                            bidx, [pvec], jnp.broadcast_to(si, (16,)),
                            mask=m0)
                        tnew = jnp.where(p <= 30, b1[14], sv)
                        tinew = jnp.where(p <= 30, i1[14], si)
                        tv = jnp.minimum(tnew, thr0)
                        ti = jnp.where(tnew <= thr0, tinew, jnp.int32(_BIG))
                        return (cm2 & (lane != pos), c_ - 1, tv, ti)

                    thr2, thri2 = lax.while_loop(wcond, wbody,
                                                 (cm, cnt, thr2, thri2))[2:]
                return (thr2, thri2)

            return lax.cond(bms <= thr_c[0], hit, lambda tc: tc, thr_c)

        lax.fori_loop(0, _NBLK, blk_body, (thr0, jnp.int32(_BIG)))

        pltpu.sync_copy(bidx.at[pl.ds(0, _K)], idx_hbm.at[rr])
        pltpu.async_copy(tab_hbm.at[bidx.at[pl.ds(0, _K)]], gbuf, gsem).wait()
        cvec = cents[rr - base_row]
        for i in range(_K):
            grows[i] = gbuf[i, pl.ds(0, _D)] - cvec
        pltpu.sync_copy(grows, out_hbm.at[pl.ds(rr * _K, _K)])

    pltpu.sync_copy(dist_hbm.at[base_row], dbuf)

    def pair_body(q, carry0):
        r0 = base_row + 2 * q
        h1 = pltpu.async_copy(dist_hbm.at[r0 + 1], dbuf2, sem2)
        scan_row(r0, dbuf)
        h1.wait()
        nxt = jnp.minimum(r0 + 2, _RR - 1)
        h2 = pltpu.async_copy(dist_hbm.at[nxt], dbuf, sem)
        scan_row(r0 + 1, dbuf2)
        h2.wait()
        return carry0

    lax.fori_loop(0, _RPW // 2, pair_body, 0)


_sc_topk = pl.kernel(
    _sc_topk_body,
    out_type=[
        jax.ShapeDtypeStruct((_RR * _K, _D), jnp.float32),
        jax.ShapeDtypeStruct((_RR, _K), jnp.int32),
    ],
    mesh=plsc.VectorSubcoreMesh(core_axis_name="c", subcore_axis_name="s"),
    compiler_params=pltpu.CompilerParams(use_tc_tiling_on_sc=False,
                                         needs_layout_passes=False),
    scratch_types=[
        pltpu.VMEM((_N,), jnp.float32),
        pltpu.VMEM((_N,), jnp.float32),
        pltpu.VMEM((_NBLK * 16,), jnp.float32),
        pltpu.VMEM((48,), jnp.float32),
        pltpu.VMEM((48,), jnp.int32),
        pltpu.VMEM((_RPW, _D), jnp.float32),
        pltpu.VMEM((_K, _TW), jnp.float32),
        pltpu.VMEM((_K, _D), jnp.float32),
        pltpu.SemaphoreType.DMA,
        pltpu.SemaphoreType.DMA,
        pltpu.SemaphoreType.DMA,
    ],
)


@jax.jit
def _run(xyz, center):
    xp = jnp.pad(xyz, ((0, 0), (0, 0), (0, 5))).transpose(0, 2, 1)
    cp = jnp.pad(center, ((0, 0), (0, 0), (0, 5)))
    dist = pl.pallas_call(_tc_dist_body, **_DIST_KW)(xp, cp)

    tab = jnp.pad(xyz.reshape(_B * _N, 3), ((0, 0), (0, _TW - 3)))
    cen = jnp.pad(center.reshape(_RR, 3), ((0, 0), (0, _D - 3)))
    out, idx = _sc_topk(dist.reshape(_RR, _N), tab, cen)
    neighborhood = out[:, :3].reshape(_B, _G, _K, 3)
    return neighborhood, idx.reshape(-1)


def kernel(xyz, center):
    neighborhood, idx_flat = _run(xyz, center)
    return (neighborhood, center, idx_flat)
